# baseline (speedup 1.0000x reference)
import functools

import jax
import jax.numpy as jnp
from jax import lax
from jax.experimental import pallas as pl
from jax.experimental.pallas import tpu as pltpu
from jax.experimental.pallas import tpu_sc as plsc

N = 16384
M = 16384
B = 64
NNZ = 262144

NC = 2
NS = 16
NW = NC * NS
RPT = N // NW
S = 512
QL = 128
Q = S // QL
LANE = 16
NSAMP = NNZ // S


mesh = plsc.VectorSubcoreMesh(core_axis_name="c", subcore_axis_name="s")


@functools.partial(
    pl.kernel,
    out_type=jax.ShapeDtypeStruct((B, NW, 8, B), jnp.float32),
    mesh=mesh,
    compiler_params=pltpu.CompilerParams(use_tc_tiling_on_sc=False,
                                         needs_layout_passes=False),
    scratch_types=[
        pltpu.VMEM((2, S + LANE), jnp.int32),
        pltpu.VMEM((2, S + LANE), jnp.int32),
        pltpu.VMEM((2, S), jnp.float32),
        pltpu.VMEM((S, B), jnp.float32),
        pltpu.VMEM((S, B), jnp.float32),
        pltpu.VMEM((RPT, B), jnp.float32),
        pltpu.SemaphoreType.DMA,
        pltpu.SemaphoreType.DMA,
        pltpu.SemaphoreType.DMA,
        pltpu.SemaphoreType.DMA,
    ],
)
def _spmm_sc(xT_hbm, wp_hbm, rowp_hbm, colp_hbm, out_hbm,
             colv, rowv, wv, buf0, buf1, acc,
             semS0, semS1, semG0, semG1):
    c = lax.axis_index("c")
    s_ax = lax.axis_index("s")
    wid = c * NS + s_ax
    lane16 = jnp.arange(LANE, dtype=jnp.int32)
    bufs = (buf0, buf1)
    semS = (semS0, semS1)
    semG = (semG0, semG1)

    def _samp_idx(r, _):
        rowv[0, pl.ds(r * LANE, LANE)] = (r * LANE + lane16) * S
        return 0
    lax.fori_loop(0, NSAMP // LANE, _samp_idx, 0)
    scps = tuple(
        pltpu.make_async_copy(rowp_hbm.at[rowv.at[0, pl.ds(g * QL, QL)]],
                              rowv.at[1, pl.ds(g * QL, QL)], semG0)
        for g in range(NSAMP // QL))
    for cp in scps:
        cp.start()

    def _zero_row(i, _):
        for j4 in range(B // LANE):
            acc[i, pl.ds(j4 * LANE, LANE)] = jnp.zeros((LANE,), jnp.float32)
        return 0
    lax.fori_loop(0, RPT, _zero_row, 0)

    for cp in scps:
        cp.wait()

    t_lo = wid * RPT
    t_hi = t_lo + RPT

    def _first_geq(bank, ref, t_val):
        def body(it, lohi):
            lo, hi = lohi
            mid = (lo + hi) // 2
            v = ref[bank, pl.ds(mid, LANE)][0]
            lt = v < t_val
            return (jnp.where(lt, mid + 1, lo), jnp.where(lt, hi, mid))
        lo, _ = lax.fori_loop(0, 9, body, (jnp.int32(0), jnp.int32(S)))
        return lo

    def _refine(cs, t_val):
        start = jnp.maximum(cs - 1, 0) * S
        pltpu.sync_copy(rowp_hbm.at[pl.ds(start, S)], colv.at[0, pl.ds(0, S)])
        return start + _first_geq(0, colv, t_val)

    s_lo = _refine(_first_geq(1, rowv, t_lo), t_lo)
    s_hi = _refine(_first_geq(1, rowv, t_hi), t_hi)

    s_al = (s_lo // QL) * QL
    n_chunks = jnp.maximum((s_hi - s_al + S - 1) // S, 1)

    def chunk_off(ci):
        off_log = s_al + ci * S
        return off_log, jnp.minimum(off_log, NNZ - S)

    def staging_copies(ci, b):
        _, off = chunk_off(ci)
        sem = semS[b]
        return (
            pltpu.make_async_copy(colp_hbm.at[pl.ds(off, S)],
                                  colv.at[b, pl.ds(0, S)], sem),
            pltpu.make_async_copy(rowp_hbm.at[pl.ds(off, S)],
                                  rowv.at[b, pl.ds(0, S)], sem),
            pltpu.make_async_copy(wp_hbm.at[pl.ds(off, S)], wv.at[b], sem),
        )

    def gather_copies(b):
        sem = semG[b]
        return tuple(
            pltpu.make_async_copy(
                xT_hbm.at[colv.at[b, pl.ds(q * QL, QL)]],
                bufs[b].at[pl.ds(q * QL, QL)], sem)
            for q in range(Q))

    def issue(copies):
        for cp in copies:
            cp.start()

    def drain(copies):
        for cp in copies:
            cp.wait()

    def scale_accum(ci, b):
        off_log, off = chunk_off(ci)
        buf = bufs[b]
        for q in range(Q):
            def grp_body(r, _, q=q):
                i0 = q * QL + r * LANE
                w16 = wv[b, pl.ds(i0, LANE)]
                r16 = rowv[b, pl.ds(i0, LANE)]
                gidx = off + i0 + lane16
                valid = ((gidx >= jnp.maximum(s_lo, off_log))
                         & (gidx < s_hi))
                w16 = jnp.where(valid, w16, jnp.float32(0.0))
                rl16 = jnp.where(valid, r16 - wid * RPT, 0)
                for half in range(2):
                    prods = []
                    for kk in range(8):
                        k = half * 8 + kk
                        wk = w16[k]
                        i = i0 + k
                        for j4 in range(B // LANE):
                            sl = pl.ds(j4 * LANE, LANE)
                            prods.append(buf[i, sl] * wk)
                    for kk in range(8):
                        k = half * 8 + kk
                        rlk = rl16[k]
                        for j4 in range(B // LANE):
                            sl = pl.ds(j4 * LANE, LANE)
                            plsc.addupdate(acc.at[rlk, sl],
                                           prods[kk * 4 + j4])
                return 0
            lax.fori_loop(0, QL // LANE, grp_body, 0)

    issue(staging_copies(0, 0))
    drain(staging_copies(0, 0))
    issue(gather_copies(0))
    issue(staging_copies(1, 1))

    def pipe_body(ci, _):
        for b in range(2):
            ci2 = ci * 2 + b
            bn = 1 - b
            drain(staging_copies(ci2 + 1, bn))
            issue(gather_copies(bn))
            drain(gather_copies(b))
            scale_accum(ci2, b)
            issue(staging_copies(ci2 + 2, b))
        return 0

    n2 = (n_chunks + 1) // 2
    lax.fori_loop(0, n2, pipe_body, 0)

    nl = n2 * 2
    drain(gather_copies(0))
    drain(staging_copies(nl + 1, 1))

    for j in range(B):
        jc = jnp.full((LANE,), j, jnp.int32)

        def tb(g4, pv, jc=jc, j=j):
            for gg in range(4):
                v = plsc.load_gather(acc, [pv, jc])
                buf0[j * 8 + g4, pl.ds(gg * LANE, LANE)] = v
                pv = pv + LANE
            return pv
        lax.fori_loop(0, 8, tb, lane16)

    wcps = tuple(
        pltpu.make_async_copy(buf0.at[pl.ds(j * 8, 8)],
                              out_hbm.at[j, wid], semG0)
        for j in range(B))
    for cp in wcps:
        cp.start()
    for cp in wcps:
        cp.wait()


@jax.jit
def kernel(input, W_val, row_ids, col_ids):
    x = input.astype(jnp.float32)
    xT = x.T
    row32 = row_ids.astype(jnp.int32)
    col32 = col_ids.astype(jnp.int32)
    out4 = _spmm_sc(xT, W_val.astype(jnp.float32), row32, col32)
    return out4.reshape(B, N)


# speedup vs baseline: 1.2254x; 1.2254x over previous
import functools

import jax
import jax.numpy as jnp
from jax import lax
from jax.experimental import pallas as pl
from jax.experimental.pallas import tpu as pltpu
from jax.experimental.pallas import tpu_sc as plsc

N = 16384
M = 16384
B = 64
NNZ = 262144

NC = 2
NS = 16
NW = NC * NS
RPT = N // NW
S = 640
QL = 128
Q = S // QL
LANE = 16
SS = 512
NSAMP = NNZ // SS


mesh = plsc.VectorSubcoreMesh(core_axis_name="c", subcore_axis_name="s")


@functools.partial(
    pl.kernel,
    out_type=jax.ShapeDtypeStruct((N, B), jnp.float32),
    mesh=mesh,
    compiler_params=pltpu.CompilerParams(use_tc_tiling_on_sc=False),
    scratch_types=[
        pltpu.VMEM((2, S + LANE), jnp.int32),
        pltpu.VMEM((2, S + LANE), jnp.int32),
        pltpu.VMEM((2, S), jnp.float32),
        pltpu.VMEM((S, B), jnp.float32),
        pltpu.VMEM((S, B), jnp.float32),
        pltpu.VMEM((RPT, B), jnp.float32),
        pltpu.SemaphoreType.DMA,
        pltpu.SemaphoreType.DMA,
        pltpu.SemaphoreType.DMA,
        pltpu.SemaphoreType.DMA,
    ],
)
def _spmm_sc(xT_hbm, wp_hbm, rowp_hbm, colp_hbm, out_hbm,
             colv, rowv, wv, buf0, buf1, acc,
             semS0, semS1, semG0, semG1):
    c = lax.axis_index("c")
    s_ax = lax.axis_index("s")
    wid = c * NS + s_ax
    lane16 = jnp.arange(LANE, dtype=jnp.int32)
    bufs = (buf0, buf1)
    semS = (semS0, semS1)
    semG = (semG0, semG1)

    def _samp_idx(r, _):
        rowv[0, pl.ds(r * LANE, LANE)] = (r * LANE + lane16) * SS
        return 0
    lax.fori_loop(0, NSAMP // LANE, _samp_idx, 0)
    scps = tuple(
        pltpu.make_async_copy(rowp_hbm.at[rowv.at[0, pl.ds(g * QL, QL)]],
                              rowv.at[1, pl.ds(g * QL, QL)], semG0)
        for g in range(NSAMP // QL))
    for cp in scps:
        cp.start()

    def _zero_row(i, _):
        for j4 in range(B // LANE):
            acc[i, pl.ds(j4 * LANE, LANE)] = jnp.zeros((LANE,), jnp.float32)
        return 0
    lax.fori_loop(0, RPT, _zero_row, 0)

    for cp in scps:
        cp.wait()

    t_lo = wid * RPT
    t_hi = t_lo + RPT

    def _first_geq(bank, ref, t_val):
        def body(it, lohi):
            lo, hi = lohi
            mid = (lo + hi) // 2
            v = ref[bank, pl.ds(mid, LANE)][0]
            lt = v < t_val
            return (jnp.where(lt, mid + 1, lo), jnp.where(lt, hi, mid))
        lo, _ = lax.fori_loop(0, 9, body, (jnp.int32(0), jnp.int32(SS)))
        return lo

    def _refine(cs, t_val):
        start = jnp.maximum(cs - 1, 0) * SS
        pltpu.sync_copy(rowp_hbm.at[pl.ds(start, SS)],
                        colv.at[0, pl.ds(0, SS)])
        return start + _first_geq(0, colv, t_val)

    s_lo = _refine(_first_geq(1, rowv, t_lo), t_lo)
    s_hi = _refine(_first_geq(1, rowv, t_hi), t_hi)

    s_al = (s_lo // QL) * QL
    n_chunks = jnp.maximum((s_hi - s_al + S - 1) // S, 1)

    def chunk_off(ci):
        off_log = s_al + ci * S
        return off_log, jnp.minimum(off_log, NNZ - S)

    def staging_copies(ci, b):
        _, off = chunk_off(ci)
        sem = semS[b]
        return (
            pltpu.make_async_copy(colp_hbm.at[pl.ds(off, S)],
                                  colv.at[b, pl.ds(0, S)], sem),
            pltpu.make_async_copy(rowp_hbm.at[pl.ds(off, S)],
                                  rowv.at[b, pl.ds(0, S)], sem),
            pltpu.make_async_copy(wp_hbm.at[pl.ds(off, S)], wv.at[b], sem),
        )

    def gather_copies(b):
        sem = semG[b]
        return tuple(
            pltpu.make_async_copy(
                xT_hbm.at[colv.at[b, pl.ds(q * QL, QL)]],
                bufs[b].at[pl.ds(q * QL, QL)], sem)
            for q in range(Q))

    def issue(copies):
        for cp in copies:
            cp.start()

    def drain(copies):
        for cp in copies:
            cp.wait()

    def scale_accum(ci, b):
        off_log, off = chunk_off(ci)
        buf = bufs[b]
        for q in range(Q):
            def grp_body(r, _, q=q):
                i0 = q * QL + r * LANE
                w16 = wv[b, pl.ds(i0, LANE)]
                r16 = rowv[b, pl.ds(i0, LANE)]
                gidx = off + i0 + lane16
                valid = ((gidx >= jnp.maximum(s_lo, off_log))
                         & (gidx < s_hi))
                w16 = jnp.where(valid, w16, jnp.float32(0.0))
                rl16 = jnp.where(valid, r16 - wid * RPT, 0)
                for half in range(2):
                    prods = []
                    for kk in range(8):
                        k = half * 8 + kk
                        wk = w16[k]
                        i = i0 + k
                        for j4 in range(B // LANE):
                            sl = pl.ds(j4 * LANE, LANE)
                            prods.append(buf[i, sl] * wk)
                    for kk in range(8):
                        k = half * 8 + kk
                        rlk = rl16[k]
                        for j4 in range(B // LANE):
                            sl = pl.ds(j4 * LANE, LANE)
                            plsc.addupdate(acc.at[rlk, sl],
                                           prods[kk * 4 + j4])
                return 0
            lax.fori_loop(0, QL // LANE, grp_body, 0)

    issue(staging_copies(0, 0))
    drain(staging_copies(0, 0))
    issue(gather_copies(0))
    issue(staging_copies(1, 1))

    def pipe_body(ci, _):
        for b in range(2):
            ci2 = ci * 2 + b
            bn = 1 - b
            drain(staging_copies(ci2 + 1, bn))
            issue(gather_copies(bn))
            drain(gather_copies(b))
            scale_accum(ci2, b)
            issue(staging_copies(ci2 + 2, b))
        return 0

    n2 = (n_chunks + 1) // 2
    lax.fori_loop(0, n2, pipe_body, 0)

    nl = n2 * 2
    drain(gather_copies(0))
    drain(staging_copies(nl + 1, 1))

    pltpu.sync_copy(acc, out_hbm.at[pl.ds(wid * RPT, RPT)])


@jax.jit
def kernel(input, W_val, row_ids, col_ids):
    x = input.astype(jnp.float32)
    xT = x.T
    row32 = row_ids.astype(jnp.int32)
    col32 = col_ids.astype(jnp.int32)
    out_nb = _spmm_sc(xT, W_val.astype(jnp.float32), row32, col32)
    return out_nb.T
